# baseline (speedup 1.0000x reference)
import functools

import jax
import jax.numpy as jnp
from jax import lax
from jax.experimental import pallas as pl
from jax.experimental.pallas import tpu as pltpu
from jax.experimental.pallas import tpu_sc as plsc

EMB_D = 32
ROW_W = 32
MID_W = 33
LANES = 16


@functools.lru_cache(maxsize=None)
def _sc_geometry():
    try:
        info = plsc.get_sparse_core_info()
        return int(info.num_cores), int(info.num_subcores)
    except Exception:
        return 2, 16


@functools.lru_cache(maxsize=None)
def _make_transpose(vocab: int):
    n_tiles = (vocab + 127) // 128
    vocab_pad = n_tiles * 128
    K = 4
    n_strips = n_tiles // K
    tail_col = n_strips * K * 128
    tail_w = vocab - tail_col
    nc, ns = _sc_geometry()
    nw = nc * ns
    W_STRIP = K * 128
    n_pairs = (n_strips // nw + 1) // 2

    mesh = plsc.VectorSubcoreMesh(core_axis_name="c", subcore_axis_name="s")

    @functools.partial(
        pl.kernel,
        mesh=mesh,
        out_type=jax.ShapeDtypeStruct((vocab_pad * ROW_W,), jnp.float32),
        scratch_types=[
            pltpu.VMEM((EMB_D, W_STRIP), jnp.float32),
            pltpu.VMEM((EMB_D, W_STRIP), jnp.float32),
            pltpu.VMEM((W_STRIP * MID_W,), jnp.float32),
            pltpu.VMEM((W_STRIP * ROW_W,), jnp.float32),
            pltpu.VMEM((W_STRIP * ROW_W,), jnp.float32),
            pltpu.VMEM((tail_w * EMB_D,), jnp.float32),
            pltpu.SemaphoreType.DMA,
            pltpu.SemaphoreType.DMA,
            pltpu.SemaphoreType.DMA,
            pltpu.SemaphoreType.DMA,
        ],
        compiler_params=pltpu.CompilerParams(use_tc_tiling_on_sc=True,
                                             needs_layout_passes=False),
    )
    def transpose_kernel(tt_hbm, tail_hbm, out_hbm, in_a, in_b, mid_v,
                         out_a, out_b, tail_v, sem_ia, sem_ib, sem_oa, sem_ob):
        wid = lax.axis_index("s") * nc + lax.axis_index("c")
        lane = lax.broadcasted_iota(jnp.int32, (LANES,), 0)
        lane_mid = lane * MID_W

        def stage(first_col, in_v, sem):
            return pltpu.async_copy(
                tt_hbm.at[:, pl.ds(first_col, W_STRIP)], in_v, sem)

        def transpose(in_v, out_v):
            def group(g, carry):
                base = g * LANES * MID_W
                col = g * LANES
                for d in range(EMB_D):
                    x = in_v[d, pl.ds(col, LANES)]
                    plsc.store_scatter(mid_v, [lane_mid + (base + d)], x)
                return carry

            lax.fori_loop(0, W_STRIP // LANES, group, 0)

            def repack(g, carry):
                for k in range(8):
                    v = g * 8 + k
                    m = v * MID_W
                    x0 = plsc.load_gather(mid_v, [lane + m])
                    x1 = plsc.load_gather(mid_v, [lane + (m + LANES)])
                    out_v[pl.ds(v * ROW_W, LANES)] = x0
                    out_v[pl.ds(v * ROW_W + LANES, LANES)] = x1
                return carry

            lax.fori_loop(0, W_STRIP // 8, repack, 0)

        def unstage(first_col, out_v, sem):
            return pltpu.async_copy(
                out_v,
                out_hbm.at[pl.ds(first_col * ROW_W, W_STRIP * ROW_W)], sem)

        def pair(j, carry):
            ta = (2 * j) * nw + wid
            tb = ta + nw
            ca = stage(ta * W_STRIP, in_a, sem_ia)

            @pl.when(tb < n_strips)
            def _():
                stage(tb * W_STRIP, in_b, sem_ib)

            ca.wait()
            transpose(in_a, out_a)
            wa = unstage(ta * W_STRIP, out_a, sem_oa)

            @pl.when(tb < n_strips)
            def _():
                pltpu.make_async_copy(
                    tt_hbm.at[:, pl.ds(0, W_STRIP)], in_b, sem_ib).wait()
                transpose(in_b, out_b)
                unstage(tb * W_STRIP, out_b, sem_ob).wait()

            wa.wait()
            return carry

        lax.fori_loop(0, n_pairs, pair, 0)

        @pl.when(wid == 0)
        def _():
            pltpu.sync_copy(tail_hbm, tail_v)
            pltpu.sync_copy(tail_v,
                            out_hbm.at[pl.ds(tail_col * ROW_W,
                                             tail_w * ROW_W)])

    return transpose_kernel


@functools.lru_cache(maxsize=None)
def _make_gather(vocab_pad: int, batch: int, chunk: int):
    nc, ns = _sc_geometry()
    nw = nc * ns
    b_per_w = batch // nw
    n_pairs = b_per_w // (2 * chunk)
    assert b_per_w % (2 * chunk) == 0 and chunk % 8 == 0

    mesh = plsc.VectorSubcoreMesh(core_axis_name="c", subcore_axis_name="s")

    @functools.partial(
        pl.kernel,
        mesh=mesh,
        out_type=jax.ShapeDtypeStruct((batch, EMB_D), jnp.float32),
        scratch_types=[
            pltpu.VMEM((chunk,), jnp.int32),
            pltpu.VMEM((chunk,), jnp.int32),
            pltpu.VMEM((chunk, ROW_W), jnp.float32),
            pltpu.VMEM((chunk, ROW_W), jnp.float32),
            pltpu.SemaphoreType.DMA,
            pltpu.SemaphoreType.DMA,
            pltpu.SemaphoreType.DMA,
            pltpu.SemaphoreType.DMA,
        ],
        compiler_params=pltpu.CompilerParams(use_tc_tiling_on_sc=False),
    )
    def gather_kernel(table_hbm, idx_hbm, out_hbm, idx_a, idx_b, rows_a, rows_b,
                      sem_ga, sem_gb, sem_wa, sem_wb):
        wid = lax.axis_index("s") * nc + lax.axis_index("c")
        base = wid * b_per_w

        def pair(j, carry):
            off_a = base + (2 * j) * chunk
            off_b = off_a + chunk
            pltpu.sync_copy(idx_hbm.at[pl.ds(off_a, chunk)], idx_a)
            ga = pltpu.async_copy(table_hbm.at[idx_a], rows_a, sem_ga)
            pltpu.sync_copy(idx_hbm.at[pl.ds(off_b, chunk)], idx_b)
            gb = pltpu.async_copy(table_hbm.at[idx_b], rows_b, sem_gb)
            ga.wait()
            wa = pltpu.async_copy(rows_a, out_hbm.at[pl.ds(off_a, chunk)], sem_wa)
            gb.wait()
            wb = pltpu.async_copy(rows_b, out_hbm.at[pl.ds(off_b, chunk)], sem_wb)
            wa.wait()
            wb.wait()
            return carry

        lax.fori_loop(0, n_pairs, pair, 0)

    return gather_kernel


def kernel(input, weight):
    b, s = input.shape
    batch = b * s
    vocab = weight.shape[0]
    vocab_pad = ((vocab + 127) // 128) * 128
    idx = input.reshape(batch).astype(jnp.int32)
    tt = jnp.swapaxes(weight, 0, 1)
    tail_col = (vocab // (4 * 128)) * 4 * 128
    tail = jnp.reshape(weight[tail_col:], (-1,))
    flat = _make_transpose(vocab)(tt, tail)
    table = jnp.reshape(flat, (vocab_pad, ROW_W))
    out = _make_gather(vocab_pad, batch, 1600)(table, idx)
    return out.reshape(b, s, EMB_D)


# speedup vs baseline: 1.4178x; 1.4178x over previous
import functools

import jax
import jax.numpy as jnp
from jax import lax
from jax.experimental import pallas as pl
from jax.experimental.pallas import tpu as pltpu
from jax.experimental.pallas import tpu_sc as plsc

EMB_D = 32
ROW_W = 40
LANES = 16


@functools.lru_cache(maxsize=None)
def _sc_geometry():
    try:
        info = plsc.get_sparse_core_info()
        return int(info.num_cores), int(info.num_subcores)
    except Exception:
        return 2, 16


@functools.lru_cache(maxsize=None)
def _make_transpose(vocab: int):
    n_tiles = (vocab + 127) // 128
    vocab_pad = n_tiles * 128
    K = 4
    n_strips = n_tiles // K
    tail_col = n_strips * K * 128
    tail_w = vocab - tail_col
    nc, ns = _sc_geometry()
    nw = nc * ns
    W_STRIP = K * 128
    n_pairs = (n_strips // nw + 1) // 2

    mesh = plsc.VectorSubcoreMesh(core_axis_name="c", subcore_axis_name="s")

    @functools.partial(
        pl.kernel,
        mesh=mesh,
        out_type=jax.ShapeDtypeStruct((vocab_pad * ROW_W,), jnp.float32),
        scratch_types=[
            pltpu.VMEM((EMB_D, W_STRIP), jnp.float32),
            pltpu.VMEM((EMB_D, W_STRIP), jnp.float32),
            pltpu.VMEM((W_STRIP * ROW_W,), jnp.float32),
            pltpu.VMEM((W_STRIP * ROW_W,), jnp.float32),
            pltpu.VMEM((tail_w * EMB_D,), jnp.float32),
            pltpu.SemaphoreType.DMA,
            pltpu.SemaphoreType.DMA,
            pltpu.SemaphoreType.DMA,
            pltpu.SemaphoreType.DMA,
        ],
        compiler_params=pltpu.CompilerParams(use_tc_tiling_on_sc=True,
                                             needs_layout_passes=False),
    )
    def transpose_kernel(tt_hbm, tail_hbm, out_hbm, in_a, in_b, out_a, out_b,
                         tail_v, sem_ia, sem_ib, sem_oa, sem_ob):
        wid = lax.axis_index("s") * nc + lax.axis_index("c")
        lane = lax.broadcasted_iota(jnp.int32, (LANES,), 0)
        lane_dst = lane * ROW_W

        def stage(first_col, in_v, sem):
            return pltpu.async_copy(
                tt_hbm.at[:, pl.ds(first_col, W_STRIP)], in_v, sem)

        def transpose(in_v, out_v):
            def group(g, carry):
                base = g * LANES * ROW_W
                col = g * LANES
                for d in range(EMB_D):
                    x = in_v[d, pl.ds(col, LANES)]
                    plsc.store_scatter(out_v, [lane_dst + (base + d)], x)
                return carry

            lax.fori_loop(0, W_STRIP // LANES, group, 0)

        def unstage(first_col, out_v, sem):
            return pltpu.async_copy(
                out_v,
                out_hbm.at[pl.ds(first_col * ROW_W, W_STRIP * ROW_W)], sem)

        def pair(j, carry):
            ta = (2 * j) * nw + wid
            tb = ta + nw
            ca = stage(ta * W_STRIP, in_a, sem_ia)

            @pl.when(tb < n_strips)
            def _():
                stage(tb * W_STRIP, in_b, sem_ib)

            ca.wait()
            transpose(in_a, out_a)
            wa = unstage(ta * W_STRIP, out_a, sem_oa)

            @pl.when(tb < n_strips)
            def _():
                pltpu.make_async_copy(
                    tt_hbm.at[:, pl.ds(0, W_STRIP)], in_b, sem_ib).wait()
                transpose(in_b, out_b)
                unstage(tb * W_STRIP, out_b, sem_ob).wait()

            wa.wait()
            return carry

        lax.fori_loop(0, n_pairs, pair, 0)

        @pl.when(wid == 0)
        def _():
            pltpu.sync_copy(tail_hbm, tail_v)
            lane_src = lane * EMB_D
            for grp in range((tail_w + LANES - 1) // LANES):
                msk = (grp * LANES + lane) < tail_w
                for d in range(EMB_D):
                    x = plsc.load_gather(
                        tail_v, [lane_src + (grp * LANES * EMB_D + d)],
                        mask=msk)
                    plsc.store_scatter(
                        out_a, [lane_dst + (grp * LANES * ROW_W + d)],
                        x, mask=msk)
            pltpu.sync_copy(out_a.at[pl.ds(0, tail_w * ROW_W)],
                            out_hbm.at[pl.ds(tail_col * ROW_W,
                                             tail_w * ROW_W)])

    return transpose_kernel


@functools.lru_cache(maxsize=None)
def _make_gather(vocab_pad: int, b_dim: int, s_dim: int, bpc: int):
    nc, ns = _sc_geometry()
    nw = nc * ns
    n_chunks = (b_dim // bpc) * s_dim
    per_w = n_chunks // nw
    seg = (bpc // 128) * 1024
    assert n_chunks % (2 * nw) == 0 and bpc % 128 == 0

    mesh = plsc.VectorSubcoreMesh(core_axis_name="c", subcore_axis_name="s")

    @functools.partial(
        pl.kernel,
        mesh=mesh,
        out_type=jax.ShapeDtypeStruct((s_dim * 4, (b_dim // 128) * 1024),
                                      jnp.float32),
        scratch_types=[
            pltpu.VMEM((bpc,), jnp.int32),
            pltpu.VMEM((bpc,), jnp.int32),
            pltpu.VMEM((bpc, ROW_W), jnp.float32),
            pltpu.VMEM((bpc, ROW_W), jnp.float32),
            pltpu.VMEM((4, seg), jnp.float32),
            pltpu.VMEM((4, seg), jnp.float32),
            pltpu.SemaphoreType.DMA,
            pltpu.SemaphoreType.DMA,
            pltpu.SemaphoreType.DMA,
            pltpu.SemaphoreType.DMA,
        ],
        compiler_params=pltpu.CompilerParams(use_tc_tiling_on_sc=False,
                                             needs_layout_passes=False),
    )
    def gather_kernel(table_hbm, idx_hbm, out_hbm, idx_a, idx_b,
                      rows_a, rows_b, tile_a, tile_b,
                      sem_ga, sem_gb, sem_wa, sem_wb):
        wid = lax.axis_index("s") * nc + lax.axis_index("c")
        mbase = wid * per_w
        lane = lax.broadcasted_iota(jnp.int32, (LANES,), 0)
        zeros = lane * 0
        qps = b_dim // bpc

        def arrange(rows_v, tile_v):
            def bgroup(g, carry):
                brow = lane + g * LANES
                coff = (g // 8) * 1024 + lax.rem(g, 8) * LANES
                for d in range(EMB_D):
                    x = plsc.load_gather(rows_v, [brow, zeros + d])
                    tile_v[d // 8, pl.ds(coff + (d % 8) * 128, LANES)] = x
                return carry

            lax.fori_loop(0, bpc // LANES, bgroup, 0)

        def dst(m):
            s = m // qps
            q = lax.rem(m, qps)
            return out_hbm.at[pl.ds(s * 4, 4), pl.ds(q * seg, seg)]

        def pair(j, carry):
            m_a = mbase + 2 * j
            m_b = m_a + 1
            off_a = m_a * bpc
            off_b = m_b * bpc
            pltpu.sync_copy(idx_hbm.at[pl.ds(off_a, bpc)], idx_a)
            ga = pltpu.async_copy(table_hbm.at[idx_a], rows_a, sem_ga)
            pltpu.sync_copy(idx_hbm.at[pl.ds(off_b, bpc)], idx_b)
            gb = pltpu.async_copy(table_hbm.at[idx_b], rows_b, sem_gb)
            ga.wait()
            arrange(rows_a, tile_a)
            wa = pltpu.async_copy(tile_a, dst(m_a), sem_wa)
            gb.wait()
            arrange(rows_b, tile_b)
            wb = pltpu.async_copy(tile_b, dst(m_b), sem_wb)
            wa.wait()
            wb.wait()
            return carry

        lax.fori_loop(0, per_w // 2, pair, 0)

    return gather_kernel


def kernel(input, weight):
    b_dim, s_dim = input.shape
    vocab = weight.shape[0]
    vocab_pad = ((vocab + 127) // 128) * 128
    idx = jnp.swapaxes(input, 0, 1).reshape(b_dim * s_dim).astype(jnp.int32)
    tt = jnp.swapaxes(weight, 0, 1)
    tail_col = (vocab // (4 * 128)) * 4 * 128
    tail = jnp.reshape(weight[tail_col:], (-1,))
    flat = _make_transpose(vocab)(tt, tail)
    table = jnp.reshape(flat, (vocab_pad, ROW_W))
    u = _make_gather(vocab_pad, b_dim, s_dim, 512)(table, idx)
    u5 = u.reshape(s_dim, 4, b_dim // 128, 8, 128)
    return u5.transpose(2, 4, 0, 1, 3).reshape(b_dim, s_dim, EMB_D)


# speedup vs baseline: 1.4966x; 1.0556x over previous
import functools

import jax
import jax.numpy as jnp
from jax import lax
from jax.experimental import pallas as pl
from jax.experimental.pallas import tpu as pltpu
from jax.experimental.pallas import tpu_sc as plsc

EMB_D = 32
ROW_W = 40
LANES = 16


@functools.lru_cache(maxsize=None)
def _sc_geometry():
    try:
        info = plsc.get_sparse_core_info()
        return int(info.num_cores), int(info.num_subcores)
    except Exception:
        return 2, 16


@functools.lru_cache(maxsize=None)
def _make_transpose(vocab: int):
    n_tiles = (vocab + 127) // 128
    vocab_pad = n_tiles * 128
    K = 4
    n_strips = n_tiles // K
    tail_col = n_strips * K * 128
    tail_w = vocab - tail_col
    nc, ns = _sc_geometry()
    nw = nc * ns
    W_STRIP = K * 128
    n_pairs = (n_strips // nw + 1) // 2

    mesh = plsc.VectorSubcoreMesh(core_axis_name="c", subcore_axis_name="s")

    @functools.partial(
        pl.kernel,
        mesh=mesh,
        out_type=jax.ShapeDtypeStruct((vocab_pad * ROW_W,), jnp.float32),
        scratch_types=[
            pltpu.VMEM((EMB_D, W_STRIP), jnp.float32),
            pltpu.VMEM((EMB_D, W_STRIP), jnp.float32),
            pltpu.VMEM((W_STRIP * ROW_W,), jnp.float32),
            pltpu.VMEM((W_STRIP * ROW_W,), jnp.float32),
            pltpu.VMEM((tail_w * EMB_D,), jnp.float32),
            pltpu.SemaphoreType.DMA,
            pltpu.SemaphoreType.DMA,
            pltpu.SemaphoreType.DMA,
            pltpu.SemaphoreType.DMA,
        ],
        compiler_params=pltpu.CompilerParams(use_tc_tiling_on_sc=True,
                                             needs_layout_passes=False),
    )
    def transpose_kernel(tt_hbm, tail_hbm, out_hbm, in_a, in_b, out_a, out_b,
                         tail_v, sem_ia, sem_ib, sem_oa, sem_ob):
        wid = lax.axis_index("s") * nc + lax.axis_index("c")
        lane = lax.broadcasted_iota(jnp.int32, (LANES,), 0)
        lane_dst = lane * ROW_W

        def stage(first_col, in_v, sem):
            return pltpu.async_copy(
                tt_hbm.at[:, pl.ds(first_col, W_STRIP)], in_v, sem)

        def transpose(in_v, out_v):
            def group(g, carry):
                base = g * LANES * ROW_W
                col = g * LANES
                for d in range(EMB_D):
                    x = in_v[d, pl.ds(col, LANES)]
                    plsc.store_scatter(out_v, [lane_dst + (base + d)], x)
                return carry

            lax.fori_loop(0, W_STRIP // LANES, group, 0)

        def unstage(first_col, out_v, sem):
            return pltpu.async_copy(
                out_v,
                out_hbm.at[pl.ds(first_col * ROW_W, W_STRIP * ROW_W)], sem)

        def drain(buf, sem):
            pltpu.make_async_copy(
                out_hbm.at[pl.ds(0, W_STRIP * ROW_W)], buf, sem).wait()

        def pair(j, carry):
            ta = (2 * j) * nw + wid
            tb = ta + nw
            stage(ta * W_STRIP, in_a, sem_ia)

            @pl.when(tb < n_strips)
            def _():
                stage(tb * W_STRIP, in_b, sem_ib)

            pltpu.make_async_copy(
                tt_hbm.at[:, pl.ds(0, W_STRIP)], in_a, sem_ia).wait()

            @pl.when(j > 0)
            def _():
                drain(out_a, sem_oa)

            transpose(in_a, out_a)
            unstage(ta * W_STRIP, out_a, sem_oa)

            @pl.when(tb < n_strips)
            def _():
                pltpu.make_async_copy(
                    tt_hbm.at[:, pl.ds(0, W_STRIP)], in_b, sem_ib).wait()

                @pl.when(j > 0)
                def _():
                    drain(out_b, sem_ob)

                transpose(in_b, out_b)
                unstage(tb * W_STRIP, out_b, sem_ob)

            return carry

        lax.fori_loop(0, n_pairs, pair, 0)
        drain(out_a, sem_oa)
        drain(out_b, sem_ob)

        @pl.when(wid == 0)
        def _():
            pltpu.sync_copy(tail_hbm, tail_v)
            lane_src = lane * EMB_D
            for grp in range((tail_w + LANES - 1) // LANES):
                msk = (grp * LANES + lane) < tail_w
                for d in range(EMB_D):
                    x = plsc.load_gather(
                        tail_v, [lane_src + (grp * LANES * EMB_D + d)],
                        mask=msk)
                    plsc.store_scatter(
                        out_a, [lane_dst + (grp * LANES * ROW_W + d)],
                        x, mask=msk)
            pltpu.sync_copy(out_a.at[pl.ds(0, tail_w * ROW_W)],
                            out_hbm.at[pl.ds(tail_col * ROW_W,
                                             tail_w * ROW_W)])

    return transpose_kernel


@functools.lru_cache(maxsize=None)
def _make_gather(vocab_pad: int, b_dim: int, s_dim: int, bpc: int):
    nc, ns = _sc_geometry()
    nw = nc * ns
    n_chunks = (b_dim // bpc) * s_dim
    per_w = n_chunks // nw
    seg = (bpc // 128) * 1024
    assert n_chunks % (2 * nw) == 0 and bpc % 128 == 0

    mesh = plsc.VectorSubcoreMesh(core_axis_name="c", subcore_axis_name="s")

    @functools.partial(
        pl.kernel,
        mesh=mesh,
        out_type=jax.ShapeDtypeStruct((s_dim * 4, (b_dim // 128) * 1024),
                                      jnp.float32),
        scratch_types=[
            pltpu.VMEM((bpc,), jnp.int32),
            pltpu.VMEM((bpc,), jnp.int32),
            pltpu.VMEM((bpc, ROW_W), jnp.float32),
            pltpu.VMEM((bpc, ROW_W), jnp.float32),
            pltpu.VMEM((4, seg), jnp.float32),
            pltpu.VMEM((4, seg), jnp.float32),
            pltpu.SemaphoreType.DMA,
            pltpu.SemaphoreType.DMA,
            pltpu.SemaphoreType.DMA,
            pltpu.SemaphoreType.DMA,
        ],
        compiler_params=pltpu.CompilerParams(use_tc_tiling_on_sc=False,
                                             needs_layout_passes=False),
    )
    def gather_kernel(table_hbm, idx_hbm, out_hbm, idx_a, idx_b,
                      rows_a, rows_b, tile_a, tile_b,
                      sem_ga, sem_gb, sem_wa, sem_wb):
        wid = lax.axis_index("s") * nc + lax.axis_index("c")
        mbase = wid * per_w
        lane = lax.broadcasted_iota(jnp.int32, (LANES,), 0)
        zeros = lane * 0
        qps = b_dim // bpc

        def arrange(rows_v, tile_v):
            def bgroup(g, carry):
                brow = lane + g * LANES
                coff = (g // 8) * 1024 + lax.rem(g, 8) * LANES
                for d in range(EMB_D):
                    x = plsc.load_gather(rows_v, [brow, zeros + d])
                    tile_v[d // 8, pl.ds(coff + (d % 8) * 128, LANES)] = x
                return carry

            lax.fori_loop(0, bpc // LANES, bgroup, 0)

        def dst(m):
            s = m // qps
            q = lax.rem(m, qps)
            return out_hbm.at[pl.ds(s * 4, 4), pl.ds(q * seg, seg)]

        def tdrain(buf, sem):
            pltpu.make_async_copy(
                out_hbm.at[pl.ds(0, 4), pl.ds(0, seg)], buf, sem).wait()

        def pair(j, carry):
            m_a = mbase + 2 * j
            m_b = m_a + 1
            off_a = m_a * bpc
            off_b = m_b * bpc
            pltpu.sync_copy(idx_hbm.at[pl.ds(off_a, bpc)], idx_a)
            ga = pltpu.async_copy(table_hbm.at[idx_a], rows_a, sem_ga)
            pltpu.sync_copy(idx_hbm.at[pl.ds(off_b, bpc)], idx_b)
            gb = pltpu.async_copy(table_hbm.at[idx_b], rows_b, sem_gb)
            ga.wait()

            @pl.when(j > 0)
            def _():
                tdrain(tile_a, sem_wa)

            arrange(rows_a, tile_a)
            pltpu.async_copy(tile_a, dst(m_a), sem_wa)
            gb.wait()

            @pl.when(j > 0)
            def _():
                tdrain(tile_b, sem_wb)

            arrange(rows_b, tile_b)
            pltpu.async_copy(tile_b, dst(m_b), sem_wb)
            return carry

        lax.fori_loop(0, per_w // 2, pair, 0)
        tdrain(tile_a, sem_wa)
        tdrain(tile_b, sem_wb)

    return gather_kernel


def kernel(input, weight):
    b_dim, s_dim = input.shape
    vocab = weight.shape[0]
    vocab_pad = ((vocab + 127) // 128) * 128
    idx = jnp.swapaxes(input, 0, 1).reshape(b_dim * s_dim).astype(jnp.int32)
    tt = jnp.swapaxes(weight, 0, 1)
    tail_col = (vocab // (4 * 128)) * 4 * 128
    tail = jnp.reshape(weight[tail_col:], (-1,))
    flat = _make_transpose(vocab)(tt, tail)
    table = jnp.reshape(flat, (vocab_pad, ROW_W))
    u = _make_gather(vocab_pad, b_dim, s_dim, 512)(table, idx)
    u5 = u.reshape(s_dim, 4, b_dim // 128, 8, 128)
    return u5.transpose(2, 4, 0, 1, 3).reshape(b_dim, s_dim, EMB_D)


# speedup vs baseline: 1.5846x; 1.0588x over previous
import functools

import jax
import jax.numpy as jnp
from jax import lax
from jax.experimental import pallas as pl
from jax.experimental.pallas import tpu as pltpu
from jax.experimental.pallas import tpu_sc as plsc

EMB_D = 32
ROW_W = 40
LANES = 16


@functools.lru_cache(maxsize=None)
def _sc_geometry():
    try:
        info = plsc.get_sparse_core_info()
        return int(info.num_cores), int(info.num_subcores)
    except Exception:
        return 2, 16


@functools.lru_cache(maxsize=None)
def _make_transpose(vocab: int):
    n_tiles = (vocab + 127) // 128
    vocab_pad = n_tiles * 128
    K = 4
    n_strips = n_tiles // K
    tail_col = n_strips * K * 128
    tail_w = vocab - tail_col
    nc, ns = _sc_geometry()
    nw = nc * ns
    W_STRIP = K * 128
    n_pairs = (n_strips // nw + 1) // 2

    mesh = plsc.VectorSubcoreMesh(core_axis_name="c", subcore_axis_name="s")

    @functools.partial(
        pl.kernel,
        mesh=mesh,
        out_type=jax.ShapeDtypeStruct((vocab_pad * ROW_W,), jnp.float32),
        scratch_types=[
            pltpu.VMEM((EMB_D, W_STRIP), jnp.float32),
            pltpu.VMEM((EMB_D, W_STRIP), jnp.float32),
            pltpu.VMEM((W_STRIP * ROW_W,), jnp.float32),
            pltpu.VMEM((W_STRIP * ROW_W,), jnp.float32),
            pltpu.VMEM((tail_w * EMB_D,), jnp.float32),
            pltpu.SemaphoreType.DMA,
            pltpu.SemaphoreType.DMA,
            pltpu.SemaphoreType.DMA,
            pltpu.SemaphoreType.DMA,
        ],
        compiler_params=pltpu.CompilerParams(use_tc_tiling_on_sc=True,
                                             needs_layout_passes=False),
    )
    def transpose_kernel(tt_hbm, tail_hbm, out_hbm, in_a, in_b, out_a, out_b,
                         tail_v, sem_ia, sem_ib, sem_oa, sem_ob):
        wid = lax.axis_index("s") * nc + lax.axis_index("c")
        lane = lax.broadcasted_iota(jnp.int32, (LANES,), 0)
        lane_dst = lane * ROW_W

        def stage(first_col, in_v, sem):
            return pltpu.async_copy(
                tt_hbm.at[:, pl.ds(first_col, W_STRIP)], in_v, sem)

        def transpose(in_v, out_v):
            def group(g, carry):
                base = g * LANES * ROW_W
                col = g * LANES
                for d in range(EMB_D):
                    x = in_v[d, pl.ds(col, LANES)]
                    plsc.store_scatter(out_v, [lane_dst + (base + d)], x)
                return carry

            lax.fori_loop(0, W_STRIP // LANES, group, 0)

        def unstage(first_col, out_v, sem):
            return pltpu.async_copy(
                out_v,
                out_hbm.at[pl.ds(first_col * ROW_W, W_STRIP * ROW_W)], sem)

        def drain(buf, sem):
            pltpu.make_async_copy(
                out_hbm.at[pl.ds(0, W_STRIP * ROW_W)], buf, sem).wait()

        def pair(j, carry):
            ta = (2 * j) * nw + wid
            tb = ta + nw
            stage(ta * W_STRIP, in_a, sem_ia)

            @pl.when(tb < n_strips)
            def _():
                stage(tb * W_STRIP, in_b, sem_ib)

            pltpu.make_async_copy(
                tt_hbm.at[:, pl.ds(0, W_STRIP)], in_a, sem_ia).wait()

            @pl.when(j > 0)
            def _():
                drain(out_a, sem_oa)

            transpose(in_a, out_a)
            unstage(ta * W_STRIP, out_a, sem_oa)

            @pl.when(tb < n_strips)
            def _():
                pltpu.make_async_copy(
                    tt_hbm.at[:, pl.ds(0, W_STRIP)], in_b, sem_ib).wait()

                @pl.when(j > 0)
                def _():
                    drain(out_b, sem_ob)

                transpose(in_b, out_b)
                unstage(tb * W_STRIP, out_b, sem_ob)

            return carry

        lax.fori_loop(0, n_pairs, pair, 0)
        drain(out_a, sem_oa)
        drain(out_b, sem_ob)

        @pl.when(wid == 0)
        def _():
            pltpu.sync_copy(tail_hbm, tail_v)
            lane_src = lane * EMB_D
            for grp in range((tail_w + LANES - 1) // LANES):
                msk = (grp * LANES + lane) < tail_w
                for d in range(EMB_D):
                    x = plsc.load_gather(
                        tail_v, [lane_src + (grp * LANES * EMB_D + d)],
                        mask=msk)
                    plsc.store_scatter(
                        out_a, [lane_dst + (grp * LANES * ROW_W + d)],
                        x, mask=msk)
            pltpu.sync_copy(out_a.at[pl.ds(0, tail_w * ROW_W)],
                            out_hbm.at[pl.ds(tail_col * ROW_W,
                                             tail_w * ROW_W)])

    return transpose_kernel


@functools.lru_cache(maxsize=None)
def _make_gather(vocab_pad: int, b_dim: int, s_dim: int, bpc: int):
    nc, ns = _sc_geometry()
    nw = nc * ns
    n_chunks = (b_dim // bpc) * s_dim
    per_w = n_chunks // nw
    seg = (bpc // 128) * 1024
    assert n_chunks % (2 * nw) == 0 and bpc % 128 == 0

    mesh = plsc.VectorSubcoreMesh(core_axis_name="c", subcore_axis_name="s")

    @functools.partial(
        pl.kernel,
        mesh=mesh,
        out_type=jax.ShapeDtypeStruct((s_dim * 4, (b_dim // 128) * 1024),
                                      jnp.float32),
        scratch_types=[
            pltpu.VMEM((bpc,), jnp.int32),
            pltpu.VMEM((bpc,), jnp.int32),
            pltpu.VMEM((bpc, ROW_W), jnp.float32),
            pltpu.VMEM((bpc, ROW_W), jnp.float32),
            pltpu.VMEM((4, seg), jnp.float32),
            pltpu.VMEM((4, seg), jnp.float32),
            pltpu.SemaphoreType.DMA,
            pltpu.SemaphoreType.DMA,
            pltpu.SemaphoreType.DMA,
            pltpu.SemaphoreType.DMA,
        ],
        compiler_params=pltpu.CompilerParams(use_tc_tiling_on_sc=False,
                                             needs_layout_passes=False),
    )
    def gather_kernel(table_hbm, idx_hbm, out_hbm, idx_a, idx_b,
                      rows_a, rows_b, tile_a, tile_b,
                      sem_ga, sem_gb, sem_wa, sem_wb):
        wid = lax.axis_index("s") * nc + lax.axis_index("c")
        mbase = wid * per_w
        lane = lax.broadcasted_iota(jnp.int32, (LANES,), 0)
        zeros = lane * 0
        qps = b_dim // bpc

        def arrange(rows_v, tile_v):
            def bgroup(g, carry):
                brow = lane + g * LANES
                coff = (g // 8) * 1024 + lax.rem(g, 8) * LANES
                for d in range(EMB_D):
                    x = plsc.load_gather(rows_v, [brow, zeros + d])
                    tile_v[d // 8, pl.ds(coff + (d % 8) * 128, LANES)] = x
                return carry

            lax.fori_loop(0, bpc // LANES, bgroup, 0)

        def dst(m):
            s = m // qps
            q = lax.rem(m, qps)
            return out_hbm.at[pl.ds(s * 4, 4), pl.ds(q * seg, seg)]

        def tdrain(buf, sem):
            pltpu.make_async_copy(
                out_hbm.at[pl.ds(0, 4), pl.ds(0, seg)], buf, sem).wait()

        def gdrain(buf, sem):
            pltpu.make_async_copy(
                table_hbm.at[pl.ds(0, bpc)], buf, sem).wait()

        def fill(m, idx_v, rows_v, sem):
            pltpu.sync_copy(idx_hbm.at[pl.ds(m * bpc, bpc)], idx_v)
            pltpu.async_copy(table_hbm.at[idx_v], rows_v, sem)

        n_pairs2 = per_w // 2

        def pair(j, carry):
            m_a = mbase + 2 * j
            m_b = m_a + 1
            gdrain(rows_a, sem_ga)

            @pl.when(j > 0)
            def _():
                tdrain(tile_a, sem_wa)

            arrange(rows_a, tile_a)

            @pl.when(j < n_pairs2 - 1)
            def _():
                fill(m_a + 2, idx_a, rows_a, sem_ga)

            pltpu.async_copy(tile_a, dst(m_a), sem_wa)
            gdrain(rows_b, sem_gb)

            @pl.when(j > 0)
            def _():
                tdrain(tile_b, sem_wb)

            arrange(rows_b, tile_b)

            @pl.when(j < n_pairs2 - 1)
            def _():
                fill(m_b + 2, idx_b, rows_b, sem_gb)

            pltpu.async_copy(tile_b, dst(m_b), sem_wb)
            return carry

        fill(mbase, idx_a, rows_a, sem_ga)
        fill(mbase + 1, idx_b, rows_b, sem_gb)
        lax.fori_loop(0, n_pairs2, pair, 0)
        tdrain(tile_a, sem_wa)
        tdrain(tile_b, sem_wb)

    return gather_kernel


def kernel(input, weight):
    b_dim, s_dim = input.shape
    vocab = weight.shape[0]
    vocab_pad = ((vocab + 127) // 128) * 128
    idx = jnp.swapaxes(input, 0, 1).reshape(b_dim * s_dim).astype(jnp.int32)
    tt = jnp.swapaxes(weight, 0, 1)
    tail_col = (vocab // (4 * 128)) * 4 * 128
    tail = jnp.reshape(weight[tail_col:], (-1,))
    flat = _make_transpose(vocab)(tt, tail)
    table = jnp.reshape(flat, (vocab_pad, ROW_W))
    u = _make_gather(vocab_pad, b_dim, s_dim, 512)(table, idx)
    u5 = u.reshape(s_dim, 4, b_dim // 128, 8, 128)
    return u5.transpose(2, 4, 0, 1, 3).reshape(b_dim, s_dim, EMB_D)
